# baseline (speedup 1.0000x reference)
import jax
import jax.numpy as jnp
from jax.experimental import pallas as pl
from jax.experimental.pallas import tpu as pltpu

_B, _A, _R, _C, _CH = 32, 3, 52, 52, 85
_CELLS = _R * _C


def _body(anch_ref, pred_ref, targ_ref, box_ref, loss_ref, acc_ref):
    i = pl.program_id(0)

    @pl.when(i == 0)
    def _init():
        acc_ref[0] = 0.0
        acc_ref[1] = 0.0

    x = pred_ref[0, :, 4:5]
    t = targ_ref[0, :, 4:5]
    bce = jnp.maximum(x, 0.0) - x * t + jnp.log1p(jnp.exp(-jnp.abs(x)))
    mask = t == 0.0
    acc_ref[0] += jnp.sum(jnp.where(mask, bce, 0.0))
    acc_ref[1] += jnp.sum(mask.astype(jnp.float32))

    a = i % 3
    w = anch_ref[a, 0]
    h = anch_ref[a, 1]
    xy = pred_ref[0, :, 0:2]
    box_ref[0, :, 0:2] = 1.0 / (1.0 + jnp.exp(-xy))
    box_ref[0, :, 2:3] = jnp.exp(pred_ref[0, :, 2:3]) * w
    box_ref[0, :, 3:4] = jnp.exp(pred_ref[0, :, 3:4]) * h

    @pl.when(i == pl.num_programs(0) - 1)
    def _fin():
        loss_ref[0] = acc_ref[0] / acc_ref[1]


def kernel(pred, target, scaled_anchors):
    n = _B * _A
    pred3 = pred.reshape(n, _CELLS, _CH)
    targ3 = target.reshape(n, _CELLS, 6)

    box, loss = pl.pallas_call(
        _body,
        grid=(n,),
        in_specs=[
            pl.BlockSpec(memory_space=pltpu.SMEM),
            pl.BlockSpec((1, _CELLS, _CH), lambda i: (i, 0, 0)),
            pl.BlockSpec((1, _CELLS, 6), lambda i: (i, 0, 0)),
        ],
        out_specs=[
            pl.BlockSpec((1, _CELLS, 4), lambda i: (i, 0, 0)),
            pl.BlockSpec(memory_space=pltpu.SMEM),
        ],
        out_shape=[
            jax.ShapeDtypeStruct((n, _CELLS, 4), jnp.float32),
            jax.ShapeDtypeStruct((1,), jnp.float32),
        ],
        scratch_shapes=[pltpu.SMEM((2,), jnp.float32)],
    )(scaled_anchors, pred3, targ3)

    return loss[0], box.reshape(_B, _A, _R, _C, 4)


# speedup vs baseline: 1.3148x; 1.3148x over previous
import jax
import jax.numpy as jnp
from jax.experimental import pallas as pl
from jax.experimental.pallas import tpu as pltpu

_B, _A, _R, _C, _CH = 32, 3, 52, 52, 85


def _body(anch_ref, pred_ref, targ_ref, box_ref, loss_ref, acc_ref):
    b = pl.program_id(0)
    a = pl.program_id(1)
    lin = b * _A + a

    @pl.when(lin == 0)
    def _init():
        acc_ref[0] = 0.0
        acc_ref[1] = 0.0

    x = pred_ref[0, 0, :, :, 4:5]
    t = targ_ref[0, 0, :, :, 4:5]
    bce = jnp.maximum(x, 0.0) - x * t + jnp.log1p(jnp.exp(-jnp.abs(x)))
    mask = t == 0.0
    acc_ref[0] += jnp.sum(jnp.where(mask, bce, 0.0))
    acc_ref[1] += jnp.sum(mask.astype(jnp.float32))

    w = anch_ref[a, 0]
    h = anch_ref[a, 1]
    xy = pred_ref[0, 0, :, :, 0:2]
    box_ref[0, 0, :, :, 0:2] = 1.0 / (1.0 + jnp.exp(-xy))
    box_ref[0, 0, :, :, 2:3] = jnp.exp(pred_ref[0, 0, :, :, 2:3]) * w
    box_ref[0, 0, :, :, 3:4] = jnp.exp(pred_ref[0, 0, :, :, 3:4]) * h

    @pl.when(lin == _B * _A - 1)
    def _fin():
        loss_ref[0] = acc_ref[0] / acc_ref[1]


def kernel(pred, target, scaled_anchors):
    box, loss = pl.pallas_call(
        _body,
        grid=(_B, _A),
        in_specs=[
            pl.BlockSpec(memory_space=pltpu.SMEM),
            pl.BlockSpec((1, 1, _R, _C, _CH), lambda b, a: (b, a, 0, 0, 0)),
            pl.BlockSpec((1, 1, _R, _C, 6), lambda b, a: (b, a, 0, 0, 0)),
        ],
        out_specs=[
            pl.BlockSpec((1, 1, _R, _C, 4), lambda b, a: (b, a, 0, 0, 0)),
            pl.BlockSpec(memory_space=pltpu.SMEM),
        ],
        out_shape=[
            jax.ShapeDtypeStruct((_B, _A, _R, _C, 4), jnp.float32),
            jax.ShapeDtypeStruct((1,), jnp.float32),
        ],
        scratch_shapes=[pltpu.SMEM((2,), jnp.float32)],
    )(scaled_anchors, pred, target)

    return loss[0], box


# speedup vs baseline: 2.0662x; 1.5715x over previous
import jax
import jax.numpy as jnp
from jax.experimental import pallas as pl
from jax.experimental.pallas import tpu as pltpu

_B, _A, _R, _C, _CH = 32, 3, 52, 52, 85


def _body(anch_ref, pred_ref, targ_ref, box_ref, loss_ref, acc_ref):
    a = pl.program_id(0)
    r = pl.program_id(1)

    @pl.when((a == 0) & (r == 0))
    def _init():
        acc_ref[0] = 0.0
        acc_ref[1] = 0.0

    x = pred_ref[0, 0, :, :, 4:5]
    t = jnp.transpose(targ_ref[0, 0, 0, :, :], (1, 0))[:, :, None]
    bce = jnp.maximum(x, 0.0) + jnp.log1p(jnp.exp(-jnp.abs(x)))
    mask = t == 0.0
    acc_ref[0] += jnp.sum(jnp.where(mask, bce, 0.0))
    acc_ref[1] += jnp.sum(mask.astype(jnp.float32))

    w = anch_ref[a, 0]
    h = anch_ref[a, 1]
    xy = pred_ref[0, 0, :, :, 0:2]
    box_ref[0, 0, :, :, 0:2] = 1.0 / (1.0 + jnp.exp(-xy))
    box_ref[0, 0, :, :, 2:3] = jnp.exp(pred_ref[0, 0, :, :, 2:3]) * w
    box_ref[0, 0, :, :, 3:4] = jnp.exp(pred_ref[0, 0, :, :, 3:4]) * h

    @pl.when((a == _A - 1) & (r == _R - 1))
    def _fin():
        loss_ref[0] = acc_ref[0] / acc_ref[1]


def kernel(pred, target, scaled_anchors):
    pred_t = jnp.transpose(pred, (1, 2, 3, 0, 4))
    targ_t = jnp.transpose(target, (1, 2, 4, 0, 3))

    box_p, loss = pl.pallas_call(
        _body,
        grid=(_A, _R),
        in_specs=[
            pl.BlockSpec(memory_space=pltpu.SMEM),
            pl.BlockSpec((1, 1, _C, _B, _CH), lambda a, r: (a, r, 0, 0, 0)),
            pl.BlockSpec((1, 1, 1, _B, _C), lambda a, r: (a, r, 4, 0, 0)),
        ],
        out_specs=[
            pl.BlockSpec((1, 1, _C, _B, 4), lambda a, r: (a, r, 0, 0, 0)),
            pl.BlockSpec(memory_space=pltpu.SMEM),
        ],
        out_shape=[
            jax.ShapeDtypeStruct((_A, _R, _C, _B, 4), jnp.float32),
            jax.ShapeDtypeStruct((1,), jnp.float32),
        ],
        scratch_shapes=[pltpu.SMEM((2,), jnp.float32)],
    )(scaled_anchors, pred_t, targ_t)

    return loss[0], jnp.transpose(box_p, (3, 0, 1, 2, 4))


# speedup vs baseline: 4.3921x; 2.1257x over previous
import jax
import jax.numpy as jnp
from jax.experimental import pallas as pl
from jax.experimental.pallas import tpu as pltpu

_B, _A, _R, _C, _CH = 32, 3, 52, 52, 85


def _body(anch_ref, pred_ref, targ_ref, box_ref, loss_ref, acc_ref):
    a = pl.program_id(0)
    r = pl.program_id(1)

    @pl.when((a == 0) & (r == 0))
    def _init():
        acc_ref[0] = 0.0
        acc_ref[1] = 0.0

    p = jnp.transpose(pred_ref[0, 0, :, :, 0:8], (2, 1, 0))

    x = p[4:5]
    t = targ_ref[0, 0, 0, :, :][None]
    bce = jnp.maximum(x, 0.0) + jnp.log1p(jnp.exp(-jnp.abs(x)))
    mask = t == 0.0
    acc_ref[0] += jnp.sum(jnp.where(mask, bce, 0.0))
    acc_ref[1] += jnp.sum(mask.astype(jnp.float32))

    w = anch_ref[a, 0]
    h = anch_ref[a, 1]
    s = 1.0 / (1.0 + jnp.exp(-p[0:2]))
    half = 4 * jax.lax.rem(r, 2)
    box_ref[:, 0, 0, half + 0, :] = s[0]
    box_ref[:, 0, 0, half + 1, :] = s[1]
    box_ref[:, 0, 0, half + 2, :] = jnp.exp(p[2]) * w
    box_ref[:, 0, 0, half + 3, :] = jnp.exp(p[3]) * h

    @pl.when((a == _A - 1) & (r == _R - 1))
    def _fin():
        loss_ref[0] = acc_ref[0] / acc_ref[1]


def kernel(pred, target, scaled_anchors):
    pred_t = jnp.transpose(pred, (1, 2, 3, 0, 4))
    targ_t = jnp.transpose(target, (1, 2, 4, 0, 3))

    box_q, loss = pl.pallas_call(
        _body,
        grid=(_A, _R),
        in_specs=[
            pl.BlockSpec(memory_space=pltpu.SMEM),
            pl.BlockSpec((1, 1, _C, _B, _CH), lambda a, r: (a, r, 0, 0, 0)),
            pl.BlockSpec((1, 1, 1, _B, _C), lambda a, r: (a, r, 4, 0, 0)),
        ],
        out_specs=[
            pl.BlockSpec((_B, 1, 1, 8, _C), lambda a, r: (0, a, r // 2, 0, 0)),
            pl.BlockSpec(memory_space=pltpu.SMEM),
        ],
        out_shape=[
            jax.ShapeDtypeStruct((_B, _A, _R // 2, 8, _C), jnp.float32),
            jax.ShapeDtypeStruct((1,), jnp.float32),
        ],
        scratch_shapes=[pltpu.SMEM((2,), jnp.float32)],
    )(scaled_anchors, pred_t, targ_t)

    box = jnp.transpose(box_q.reshape(_B, _A, _R, 4, _C), (0, 1, 2, 4, 3))
    return loss[0], box


# speedup vs baseline: 6.5421x; 1.4895x over previous
import jax
import jax.numpy as jnp
from jax.experimental import pallas as pl
from jax.experimental.pallas import tpu as pltpu

_B, _A, _R, _C, _CH = 32, 3, 52, 52, 85
_RB = 2


def _body(anch_ref, pred_ref, targ_ref, box_ref, loss_ref, acc_ref):
    a = pl.program_id(0)
    j = pl.program_id(1)

    @pl.when((a == 0) & (j == 0))
    def _init():
        acc_ref[0] = 0.0
        acc_ref[1] = 0.0

    w = anch_ref[a, 0]
    h = anch_ref[a, 1]
    s_bce = 0.0
    s_cnt = 0.0
    for k in range(_RB):
        p = jnp.transpose(pred_ref[0, k, :, :, 0:8], (2, 1, 0))

        x = p[4:5]
        t = targ_ref[0, k, 0, :, :][None]
        bce = jnp.maximum(x, 0.0) + jnp.log1p(jnp.exp(-jnp.abs(x)))
        mask = t == 0.0
        s_bce += jnp.sum(jnp.where(mask, bce, 0.0))
        s_cnt += jnp.sum(mask.astype(jnp.float32))

        s = 1.0 / (1.0 + jnp.exp(-p[0:2]))
        box_ref[:, 0, 0, 4 * k + 0, :] = s[0]
        box_ref[:, 0, 0, 4 * k + 1, :] = s[1]
        box_ref[:, 0, 0, 4 * k + 2, :] = jnp.exp(p[2]) * w
        box_ref[:, 0, 0, 4 * k + 3, :] = jnp.exp(p[3]) * h

    acc_ref[0] += s_bce
    acc_ref[1] += s_cnt

    @pl.when((a == _A - 1) & (j == _R // _RB - 1))
    def _fin():
        loss_ref[0] = acc_ref[0] / acc_ref[1]


def kernel(pred, target, scaled_anchors):
    pred_t = jnp.transpose(pred, (1, 2, 3, 0, 4))
    targ_t = jnp.transpose(target, (1, 2, 4, 0, 3))

    box_q, loss = pl.pallas_call(
        _body,
        grid=(_A, _R // _RB),
        in_specs=[
            pl.BlockSpec(memory_space=pltpu.SMEM),
            pl.BlockSpec((1, _RB, _C, _B, _CH), lambda a, j: (a, j, 0, 0, 0)),
            pl.BlockSpec((1, _RB, 1, _B, _C), lambda a, j: (a, j, 4, 0, 0)),
        ],
        out_specs=[
            pl.BlockSpec((_B, 1, 1, 8, _C), lambda a, j: (0, a, j, 0, 0)),
            pl.BlockSpec(memory_space=pltpu.SMEM),
        ],
        out_shape=[
            jax.ShapeDtypeStruct((_B, _A, _R // 2, 8, _C), jnp.float32),
            jax.ShapeDtypeStruct((1,), jnp.float32),
        ],
        scratch_shapes=[pltpu.SMEM((2,), jnp.float32)],
    )(scaled_anchors, pred_t, targ_t)

    box = jnp.transpose(box_q.reshape(_B, _A, _R, 4, _C), (0, 1, 2, 4, 3))
    return loss[0], box


# speedup vs baseline: 8.5075x; 1.3004x over previous
import jax
import jax.numpy as jnp
from jax.experimental import pallas as pl
from jax.experimental.pallas import tpu as pltpu

_B, _A, _R, _C, _CH = 32, 3, 52, 52, 85
_RB = 4


def _body(anch_ref, pred_ref, targ_ref, box_ref, loss_ref, acc_ref):
    a = pl.program_id(0)
    j = pl.program_id(1)

    @pl.when((a == 0) & (j == 0))
    def _init():
        acc_ref[0] = 0.0
        acc_ref[1] = 0.0

    w = anch_ref[a, 0]
    h = anch_ref[a, 1]
    s_bce = 0.0
    s_cnt = 0.0
    for k in range(_RB):
        p = jnp.transpose(pred_ref[0, k, :, :, 0:8], (2, 1, 0))

        x = p[4:5]
        t = targ_ref[0, k, 0, :, :][None]
        bce = jnp.maximum(x, 0.0) + jnp.log1p(jnp.exp(-jnp.abs(x)))
        mask = t == 0.0
        s_bce += jnp.sum(jnp.where(mask, bce, 0.0))
        s_cnt += jnp.sum(mask.astype(jnp.float32))

        s = 1.0 / (1.0 + jnp.exp(-p[0:2]))
        box_ref[:, 0, k // 2, 4 * (k % 2) + 0, :] = s[0]
        box_ref[:, 0, k // 2, 4 * (k % 2) + 1, :] = s[1]
        box_ref[:, 0, k // 2, 4 * (k % 2) + 2, :] = jnp.exp(p[2]) * w
        box_ref[:, 0, k // 2, 4 * (k % 2) + 3, :] = jnp.exp(p[3]) * h

    acc_ref[0] += s_bce
    acc_ref[1] += s_cnt

    @pl.when((a == _A - 1) & (j == _R // _RB - 1))
    def _fin():
        loss_ref[0] = acc_ref[0] / acc_ref[1]


def kernel(pred, target, scaled_anchors):
    pred_t = jnp.transpose(pred, (1, 2, 3, 0, 4))
    targ_t = jnp.transpose(target, (1, 2, 4, 0, 3))

    box_q, loss = pl.pallas_call(
        _body,
        grid=(_A, _R // _RB),
        in_specs=[
            pl.BlockSpec(memory_space=pltpu.SMEM),
            pl.BlockSpec((1, _RB, _C, _B, _CH), lambda a, j: (a, j, 0, 0, 0)),
            pl.BlockSpec((1, _RB, 1, _B, _C), lambda a, j: (a, j, 4, 0, 0)),
        ],
        out_specs=[
            pl.BlockSpec((_B, 1, _RB // 2, 8, _C), lambda a, j: (0, a, j, 0, 0)),
            pl.BlockSpec(memory_space=pltpu.SMEM),
        ],
        out_shape=[
            jax.ShapeDtypeStruct((_B, _A, _R // 2, 8, _C), jnp.float32),
            jax.ShapeDtypeStruct((1,), jnp.float32),
        ],
        scratch_shapes=[pltpu.SMEM((2,), jnp.float32)],
    )(scaled_anchors, pred_t, targ_t)

    box = jnp.transpose(box_q.reshape(_B, _A, _R, 4, _C), (0, 1, 2, 4, 3))
    return loss[0], box


# speedup vs baseline: 10.3381x; 1.2152x over previous
import jax
import jax.numpy as jnp
from jax.experimental import pallas as pl
from jax.experimental.pallas import tpu as pltpu

_B, _A, _R, _C, _CH = 32, 3, 52, 52, 85
_RB = 13


def _body(anch_ref, pred_ref, targ_ref, box_ref, loss_ref, acc_ref):
    a = pl.program_id(0)
    j = pl.program_id(1)

    @pl.when((a == 0) & (j == 0))
    def _init():
        acc_ref[0] = 0.0
        acc_ref[1] = 0.0

    w = anch_ref[a, 0]
    h = anch_ref[a, 1]
    s_bce = 0.0
    s_cnt = 0.0
    for k in range(_RB):
        p = jnp.transpose(pred_ref[0, k, :, :, 0:8], (2, 1, 0))

        x = p[4:5]
        t = targ_ref[0, k, 0, :, :][None]
        bce = jnp.maximum(x, 0.0) + jnp.log1p(jnp.exp(-jnp.abs(x)))
        mask = t == 0.0
        s_bce += jnp.sum(jnp.where(mask, bce, 0.0))
        s_cnt += jnp.sum(mask.astype(jnp.float32))

        s = 1.0 / (1.0 + jnp.exp(-p[0:2]))
        row = j * _RB + k
        r2 = row // 2
        half = 4 * (row % 2)
        box_ref[:, 0, r2, half + 0, :] = s[0]
        box_ref[:, 0, r2, half + 1, :] = s[1]
        box_ref[:, 0, r2, half + 2, :] = jnp.exp(p[2]) * w
        box_ref[:, 0, r2, half + 3, :] = jnp.exp(p[3]) * h

    acc_ref[0] += s_bce
    acc_ref[1] += s_cnt

    @pl.when((a == _A - 1) & (j == _R // _RB - 1))
    def _fin():
        loss_ref[0] = acc_ref[0] / acc_ref[1]


def kernel(pred, target, scaled_anchors):
    pred_t = jnp.transpose(pred, (1, 2, 3, 0, 4))
    targ_t = jnp.transpose(target, (1, 2, 4, 0, 3))

    box_q, loss = pl.pallas_call(
        _body,
        grid=(_A, _R // _RB),
        in_specs=[
            pl.BlockSpec(memory_space=pltpu.SMEM),
            pl.BlockSpec((1, _RB, _C, _B, _CH), lambda a, j: (a, j, 0, 0, 0)),
            pl.BlockSpec((1, _RB, 1, _B, _C), lambda a, j: (a, j, 4, 0, 0)),
        ],
        out_specs=[
            pl.BlockSpec((_B, 1, _R // 2, 8, _C), lambda a, j: (0, a, 0, 0, 0)),
            pl.BlockSpec(memory_space=pltpu.SMEM),
        ],
        out_shape=[
            jax.ShapeDtypeStruct((_B, _A, _R // 2, 8, _C), jnp.float32),
            jax.ShapeDtypeStruct((1,), jnp.float32),
        ],
        scratch_shapes=[pltpu.SMEM((2,), jnp.float32)],
    )(scaled_anchors, pred_t, targ_t)

    box = jnp.transpose(box_q.reshape(_B, _A, _R, 4, _C), (0, 1, 2, 4, 3))
    return loss[0], box
